# baseline (speedup 1.0000x reference)
import functools

import jax
import jax.numpy as jnp
from jax import lax
from jax.experimental import pallas as pl
from jax.experimental.pallas import tpu as pltpu
from jax.experimental.pallas import tpu_sc as plsc

_CH = 128


@functools.cache
def _make_kernel(B, D, L, NC, NS):
    NW = NC * NS
    b_per_w = B // NW
    NCH = b_per_w // _CH
    JU = D // L
    mesh = plsc.VectorSubcoreMesh(core_axis_name="c", subcore_axis_name="s")

    @functools.partial(
        pl.kernel,
        mesh=mesh,
        out_type=jax.ShapeDtypeStruct((NW, L), jnp.float32),
        scratch_types=[
            pltpu.VMEM((b_per_w,), jnp.int32),
            pltpu.VMEM((4, _CH, D), jnp.float32),
            pltpu.VMEM((2, _CH, D), jnp.float32),
            pltpu.VMEM((L,), jnp.float32),
            pltpu.SemaphoreType.DMA,
            pltpu.SemaphoreType.DMA,
            pltpu.SemaphoreType.DMA,
            pltpu.SemaphoreType.DMA,
            pltpu.SemaphoreType.DMA,
            pltpu.SemaphoreType.DMA,
        ],
    )
    def k(label_hbm, feat_hbm, centers_hbm, out_hbm, idx_v, rows_v, feat_v,
          part_v, sem_f0, sem_f1, sem_g0, sem_g1, sem_g2, sem_g3):
        wid = lax.axis_index("s") * NC + lax.axis_index("c")
        base = wid * b_per_w
        gsems = (sem_g0, sem_g1, sem_g2, sem_g3)
        fsems = (sem_f0, sem_f1)

        def start_feat(c):
            return pltpu.async_copy(
                feat_hbm.at[pl.ds(base + c * _CH, _CH)],
                feat_v.at[c % 2], fsems[c % 2])

        pltpu.sync_copy(label_hbm.at[pl.ds(base, b_per_w)], idx_v)
        gathers = [pltpu.async_copy(
            centers_hbm.at[idx_v.at[pl.ds(c * _CH, _CH)]],
            rows_v.at[c], gsems[c]) for c in range(NCH)]
        feats = [start_feat(0), start_feat(1)]

        accs = tuple(jnp.zeros((L,), jnp.float32) for _ in range(2 * JU))
        for c in range(NCH):
            with jax.named_scope(f"wait{c}"):
                gathers[c].wait()
                feats[c].wait()
            if c + 2 < NCH:
                feats.append(start_feat(c + 2))
            fbuf = c % 2

            def row_body(i, accs, c=c, fbuf=fbuf):
                i2 = i * 2
                new = []
                for u in range(2):
                    for j in range(JU):
                        f = feat_v[fbuf, i2 + u, pl.ds(j * L, L)]
                        r = rows_v[c, i2 + u, pl.ds(j * L, L)]
                        d = f - r
                        new.append(accs[u * JU + j] + d * d)
                return tuple(new)

            with jax.named_scope(f"cmp{c}"):
                accs = lax.fori_loop(0, _CH // 2, row_body, accs)

        tot = accs[0]
        for j in range(1, 2 * JU):
            tot = tot + accs[j]
        part_v[...] = tot
        pltpu.sync_copy(part_v, out_hbm.at[wid])

    return k


def kernel(label, feat, centers):
    B, D = feat.shape
    info = plsc.get_sparse_core_info()
    k = _make_kernel(B, D, info.num_lanes, info.num_cores, info.num_subcores)
    partials = k(label, feat, centers)
    return jnp.sum(partials) / (2.0 * B)


# speedup vs baseline: 1.0321x; 1.0321x over previous
import functools

import jax
import jax.numpy as jnp
from jax import lax
from jax.experimental import pallas as pl
from jax.experimental.pallas import tpu as pltpu
from jax.experimental.pallas import tpu_sc as plsc

_CH = 64
_FB = 4


@functools.cache
def _make_kernel(B, D, L, NC, NS):
    NW = NC * NS
    b_per_w = B // NW
    NCH = b_per_w // _CH
    JU = D // L
    mesh = plsc.VectorSubcoreMesh(core_axis_name="c", subcore_axis_name="s")

    @functools.partial(
        pl.kernel,
        mesh=mesh,
        out_type=jax.ShapeDtypeStruct((NW, L), jnp.float32),
        scratch_types=[
            pltpu.VMEM((b_per_w,), jnp.int32),
            pltpu.VMEM((b_per_w // _CH, _CH, D), jnp.float32),
            pltpu.VMEM((_FB, _CH, D), jnp.float32),
            pltpu.VMEM((L,), jnp.float32),
            pltpu.SemaphoreType.DMA,
            pltpu.SemaphoreType.DMA,
        ],
    )
    def k(label_hbm, feat_hbm, centers_hbm, out_hbm, idx_v, rows_v, feat_v,
          part_v, sem_g, sem_f):
        wid = lax.axis_index("s") * NC + lax.axis_index("c")
        base = wid * b_per_w

        def start_feat(c):
            return pltpu.async_copy(
                feat_hbm.at[pl.ds(base + c * _CH, _CH)],
                feat_v.at[c % _FB], sem_f)

        pltpu.sync_copy(label_hbm.at[pl.ds(base, b_per_w)], idx_v)
        gathers = [pltpu.async_copy(
            centers_hbm.at[idx_v.at[pl.ds(c * _CH, _CH)]],
            rows_v.at[c], sem_g) for c in range(NCH)]
        feats = [start_feat(c) for c in range(_FB)]

        accs = tuple(jnp.zeros((L,), jnp.float32) for _ in range(2 * JU))
        for c in range(NCH):
            gathers[c].wait()
            feats[c].wait()
            if c + _FB < NCH:
                feats.append(start_feat(c + _FB))
            fbuf = c % _FB

            def row_body(i, accs, c=c, fbuf=fbuf):
                i2 = i * 2
                new = []
                for u in range(2):
                    for j in range(JU):
                        f = feat_v[fbuf, i2 + u, pl.ds(j * L, L)]
                        r = rows_v[c, i2 + u, pl.ds(j * L, L)]
                        d = f - r
                        new.append(accs[u * JU + j] + d * d)
                return tuple(new)

            accs = lax.fori_loop(0, _CH // 2, row_body, accs)

        tot = accs[0]
        for j in range(1, 2 * JU):
            tot = tot + accs[j]
        part_v[...] = tot
        pltpu.sync_copy(part_v, out_hbm.at[wid])

    return k


def kernel(label, feat, centers):
    B, D = feat.shape
    info = plsc.get_sparse_core_info()
    k = _make_kernel(B, D, info.num_lanes, info.num_cores, info.num_subcores)
    partials = k(label, feat, centers)
    return jnp.sum(partials) / (2.0 * B)


# speedup vs baseline: 1.0402x; 1.0078x over previous
import functools

import jax
import jax.numpy as jnp
from jax import lax
from jax.experimental import pallas as pl
from jax.experimental.pallas import tpu as pltpu
from jax.experimental.pallas import tpu_sc as plsc

_CH = 64
_FB = 4


@functools.cache
def _make_kernel(B, D, L, NC, NS):
    NW = NC * NS
    b_per_w = B // NW
    NCH = b_per_w // _CH
    JU = D // L
    mesh = plsc.VectorSubcoreMesh(core_axis_name="c", subcore_axis_name="s")

    @functools.partial(
        pl.kernel,
        mesh=mesh,
        out_type=jax.ShapeDtypeStruct((NW, L), jnp.float32),
        scratch_types=[
            pltpu.VMEM((b_per_w,), jnp.int32),
            pltpu.VMEM((b_per_w // _CH, _CH, D), jnp.float32),
            pltpu.VMEM((_FB, _CH, D), jnp.float32),
            pltpu.VMEM((L,), jnp.float32),
            pltpu.SemaphoreType.DMA,
            pltpu.SemaphoreType.DMA,
        ],
    )
    def k(label_hbm, feat_hbm, centers_hbm, out_hbm, idx_v, rows_v, feat_v,
          part_v, sem_g, sem_f):
        wid = lax.axis_index("s") * NC + lax.axis_index("c")
        base = wid * b_per_w

        def start_feat(c):
            return pltpu.async_copy(
                feat_hbm.at[pl.ds(base + c * _CH, _CH)],
                feat_v.at[c % _FB], sem_f)

        pltpu.sync_copy(label_hbm.at[pl.ds(base, b_per_w)], idx_v)

        def start_gather(c):
            return pltpu.async_copy(
                centers_hbm.at[idx_v.at[pl.ds(c * _CH, _CH)]],
                rows_v.at[c], sem_g)

        depth = 2
        gathers = [start_gather(c) for c in range(depth)]
        feats = [start_feat(c) for c in range(depth)]

        accs = tuple(jnp.zeros((L,), jnp.float32) for _ in range(2 * JU))
        for c in range(NCH):
            if c + depth < NCH:
                gathers.append(start_gather(c + depth))
                feats.append(start_feat(c + depth))
            gathers[c].wait()
            feats[c].wait()
            fbuf = c % _FB

            def row_body(i, accs, c=c, fbuf=fbuf):
                i2 = i * 2
                new = []
                for u in range(2):
                    for j in range(JU):
                        f = feat_v[fbuf, i2 + u, pl.ds(j * L, L)]
                        r = rows_v[c, i2 + u, pl.ds(j * L, L)]
                        d = f - r
                        new.append(accs[u * JU + j] + d * d)
                return tuple(new)

            accs = lax.fori_loop(0, _CH // 2, row_body, accs)

        tot = accs[0]
        for j in range(1, 2 * JU):
            tot = tot + accs[j]
        part_v[...] = tot
        pltpu.sync_copy(part_v, out_hbm.at[wid])

    return k


def kernel(label, feat, centers):
    B, D = feat.shape
    info = plsc.get_sparse_core_info()
    k = _make_kernel(B, D, info.num_lanes, info.num_cores, info.num_subcores)
    partials = k(label, feat, centers)
    return jnp.sum(partials) / (2.0 * B)


# speedup vs baseline: 1.0472x; 1.0067x over previous
import functools

import jax
import jax.numpy as jnp
from jax import lax
from jax.experimental import pallas as pl
from jax.experimental.pallas import tpu as pltpu
from jax.experimental.pallas import tpu_sc as plsc

_CH = 64
_FB = 4


@functools.cache
def _make_kernel(B, D, L, NC, NS):
    NW = NC * NS
    b_per_w = B // NW
    NCH = b_per_w // _CH
    JU = D // L
    mesh = plsc.VectorSubcoreMesh(core_axis_name="c", subcore_axis_name="s")

    @functools.partial(
        pl.kernel,
        mesh=mesh,
        out_type=jax.ShapeDtypeStruct((NW, L), jnp.float32),
        scratch_types=[
            pltpu.VMEM((b_per_w,), jnp.int32),
            pltpu.VMEM((b_per_w // _CH, _CH, D), jnp.float32),
            pltpu.VMEM((_FB, _CH, D), jnp.float32),
            pltpu.VMEM((L,), jnp.float32),
            pltpu.SemaphoreType.DMA,
            pltpu.SemaphoreType.DMA,
        ],
    )
    def k(label_hbm, feat_hbm, centers_hbm, out_hbm, idx_v, rows_v, feat_v,
          part_v, sem_g, sem_f):
        wid = lax.axis_index("s") * NC + lax.axis_index("c")
        base = wid * b_per_w

        def start_feat(c):
            return pltpu.async_copy(
                feat_hbm.at[pl.ds(base + c * _CH, _CH)],
                feat_v.at[c % _FB], sem_f)

        pltpu.sync_copy(label_hbm.at[pl.ds(base, b_per_w)], idx_v)

        def start_gather(c):
            return pltpu.async_copy(
                centers_hbm.at[idx_v.at[pl.ds(c * _CH, _CH)]],
                rows_v.at[c], sem_g)

        depth = 2
        gathers = [start_gather(c) for c in range(depth)]
        feats = [start_feat(c) for c in range(depth)]

        accs = tuple(jnp.zeros((L,), jnp.float32) for _ in range(2 * JU))
        for c in range(NCH):
            if c + depth < NCH:
                gathers.append(start_gather(c + depth))
                feats.append(start_feat(c + depth))
            with jax.named_scope(f"wait{c}"):
                gathers[c].wait()
                feats[c].wait()
            fbuf = c % _FB

            def row_body(i, accs, c=c, fbuf=fbuf):
                i2 = i * 2
                new = []
                for u in range(2):
                    for j in range(JU):
                        f = feat_v[fbuf, i2 + u, pl.ds(j * L, L)]
                        r = rows_v[c, i2 + u, pl.ds(j * L, L)]
                        d = f - r
                        new.append(accs[u * JU + j] + d * d)
                return tuple(new)

            with jax.named_scope(f"cmp{c}"):
                accs = lax.fori_loop(0, _CH // 2, row_body, accs)

        tot = accs[0]
        for j in range(1, 2 * JU):
            tot = tot + accs[j]
        part_v[...] = tot
        pltpu.sync_copy(part_v, out_hbm.at[wid])

    return k


def kernel(label, feat, centers):
    B, D = feat.shape
    info = plsc.get_sparse_core_info()
    k = _make_kernel(B, D, info.num_lanes, info.num_cores, info.num_subcores)
    partials = k(label, feat, centers)
    return jnp.sum(partials) / (2.0 * B)
